# Initial kernel scaffold; baseline (speedup 1.0000x reference)
import jax
import jax.numpy as jnp
from jax.experimental import pallas as pl


def kernel(x, edge_index, batch, W1, b1, W2, b2, fc_W, fc_b):
    raise NotImplementedError("write your pallas kernel here")


# speedup vs baseline: 1.3225x; 1.3225x over previous
import functools

import jax
import jax.numpy as jnp
from jax import lax
from jax.experimental import pallas as pl
from jax.experimental.pallas import tpu as pltpu
from jax.experimental.pallas import tpu_sc as plsc

N_NODES = 10000
N_EDGES = 160000
N_GRAPHS = 128
MAX_NODES = 256
D = 300
H = 768
L = 5

DP = 384
D2P = 608
NP = 10496
ZERO_ROW = 10200
EB = 64
NC = 2
NS = 16
NT = NC * NS
PH = 2
ROWS_PH = NP // (NT * PH)
NBLK = NT * PH
BLK = 328


def _agg_body(x_hbm, srcs_hbm, dsts_hbm, meta_hbm, zrows_hbm, agg_hbm,
              mvec, sidx, didx, gidx, ldix, stage, acc, sem):
    c = lax.axis_index("c")
    s = lax.axis_index("s")
    t = s * NC + c

    pltpu.sync_copy(meta_hbm.at[t], mvec)
    m = mvec[...]
    lane = lax.broadcasted_iota(jnp.int32, (16,), 0)

    for p in range(PH):
        start = m[2 * p]
        cnt = m[2 * p + 1]
        qbase = (t * PH + p) * ROWS_PH

        pltpu.sync_copy(zrows_hbm, acc)

        nb = (cnt + EB - 1) // EB

        def body(i, carry):
            off = pl.multiple_of(start + i * EB, 8)
            pltpu.sync_copy(srcs_hbm.at[pl.ds(off, EB)], sidx)
            pltpu.sync_copy(dsts_hbm.at[pl.ds(off, EB)], didx)
            for j in range(EB // 16):
                sv = sidx[pl.ds(j * 16, 16)]
                dv = didx[pl.ds(j * 16, 16)]
                eid = i * EB + j * 16 + lane
                ok = (eid < cnt) & (dv >= qbase) & (dv < qbase + ROWS_PH)
                gidx[pl.ds(j * 16, 16)] = jnp.where(ok, sv, ZERO_ROW)
                ldix[pl.ds(j * 16, 16)] = jnp.where(ok, dv - qbase, 0)
            pltpu.async_copy(x_hbm.at[gidx], stage, sem).wait()

            def edge(e, carry2):
                lv = ldix[pl.ds(e, 16)]
                rb = lv[0] * DP
                for cc in range(DP // 16):
                    vals = stage[e, pl.ds(cc * 16, 16)]
                    plsc.addupdate(acc.at[pl.ds(rb + cc * 16, 16)], vals)
                return carry2

            lax.fori_loop(0, EB, edge, 0)
            return carry

        lax.fori_loop(0, nb, body, 0)
        pltpu.sync_copy(acc, agg_hbm.at[pl.ds(qbase * DP, ROWS_PH * DP)])


@functools.lru_cache(maxsize=1)
def _make_agg():
    return pl.kernel(
        _agg_body,
        out_type=jax.ShapeDtypeStruct((NP * DP,), jnp.float32),
        mesh=plsc.VectorSubcoreMesh(core_axis_name="c", subcore_axis_name="s"),
        compiler_params=pltpu.CompilerParams(needs_layout_passes=False),
        scratch_types=[
            pltpu.VMEM((16,), jnp.int32),
            pltpu.VMEM((EB,), jnp.int32),
            pltpu.VMEM((EB,), jnp.int32),
            pltpu.VMEM((EB,), jnp.int32),
            pltpu.VMEM((EB + 16,), jnp.int32),
            pltpu.VMEM((EB, DP), jnp.float32),
            pltpu.VMEM((ROWS_PH * DP,), jnp.float32),
            pltpu.SemaphoreType.DMA,
        ],
    )


def _agg(h, srcs_p, dsts_p, meta, zrows):
    return _make_agg()(h, srcs_p, dsts_p, meta, zrows)


def _mlp_body(h_ref, a_ref, w1_ref, b1_ref, w2_ref, b2_ref, o_ref, *, relu_out):
    i = pl.program_id(0)
    z = h_ref[...] + a_ref[...]
    hid = jnp.maximum(
        jnp.dot(z, w1_ref[...], preferred_element_type=jnp.float32) + b1_ref[...],
        0.0)
    out = jnp.dot(hid, w2_ref[...], preferred_element_type=jnp.float32) + b2_ref[...]
    if relu_out:
        out = jnp.maximum(out, 0.0)
    rows = i * BLK + lax.broadcasted_iota(jnp.int32, (BLK, 1), 0)
    o_ref[...] = jnp.where(rows < N_NODES, out, 0.0)


def _mlp(h, agg, w1, b1, w2, b2, relu_out):
    return pl.pallas_call(
        functools.partial(_mlp_body, relu_out=relu_out),
        grid=(NP // BLK,),
        in_specs=[
            pl.BlockSpec((BLK, DP), lambda i: (i, 0)),
            pl.BlockSpec((BLK, DP), lambda i: (i, 0)),
            pl.BlockSpec((DP, D2P), lambda i: (0, 0)),
            pl.BlockSpec((1, D2P), lambda i: (0, 0)),
            pl.BlockSpec((D2P, DP), lambda i: (0, 0)),
            pl.BlockSpec((1, DP), lambda i: (0, 0)),
        ],
        out_specs=pl.BlockSpec((BLK, DP), lambda i: (i, 0)),
        out_shape=jax.ShapeDtypeStruct((NP, DP), jnp.float32),
    )(h, agg, w1, b1, w2, b2)


def _fc_body(h_ref, w_ref, b_ref, o_ref):
    o_ref[...] = (
        jnp.dot(h_ref[...], w_ref[...], preferred_element_type=jnp.float32)
        + b_ref[...])


def _fc(h, fc_w, fc_b2):
    return pl.pallas_call(
        _fc_body,
        grid=(NP // BLK,),
        in_specs=[
            pl.BlockSpec((BLK, DP), lambda i: (i, 0)),
            pl.BlockSpec((DP, H), lambda i: (0, 0)),
            pl.BlockSpec((1, H), lambda i: (0, 0)),
        ],
        out_specs=pl.BlockSpec((BLK, H), lambda i: (i, 0)),
        out_shape=jax.ShapeDtypeStruct((NP, H), jnp.float32),
    )(h, fc_w, fc_b2)


NSLOT = N_GRAPHS * MAX_NODES
SLOT_T = NSLOT // NT
OB = 64


def _batch_body(y_hbm, map_hbm, out_hbm, midx, stage, sem):
    c = lax.axis_index("c")
    s = lax.axis_index("s")
    t = s * NC + c

    def body(b, carry):
        base = pl.multiple_of(t * SLOT_T + b * OB, 8)
        pltpu.sync_copy(map_hbm.at[pl.ds(base, OB)], midx)
        pltpu.async_copy(y_hbm.at[midx], stage, sem).wait()
        pltpu.sync_copy(stage, out_hbm.at[pl.ds(base, OB)])
        return carry

    lax.fori_loop(0, SLOT_T // OB, body, 0)


@functools.lru_cache(maxsize=1)
def _make_batch():
    return pl.kernel(
        _batch_body,
        out_type=jax.ShapeDtypeStruct((NSLOT, H), jnp.float32),
        mesh=plsc.VectorSubcoreMesh(core_axis_name="c", subcore_axis_name="s"),
        compiler_params=pltpu.CompilerParams(needs_layout_passes=False),
        scratch_types=[
            pltpu.VMEM((OB,), jnp.int32),
            pltpu.VMEM((OB, H), jnp.float32),
            pltpu.SemaphoreType.DMA,
        ],
    )


def _dense_out(src_map, y):
    return _make_batch()(y, src_map).reshape(N_GRAPHS, MAX_NODES, H)


def kernel(x, edge_index, batch, W1, b1, W2, b2, fc_W, fc_b):
    src = edge_index[0].astype(jnp.int32)
    dst = edge_index[1].astype(jnp.int32)
    perm = jnp.argsort(dst)
    src_s = src[perm]
    dst_s = dst[perm]
    bb = jnp.searchsorted(
        dst_s, jnp.arange(NBLK + 1, dtype=jnp.int32) * ROWS_PH).astype(jnp.int32)
    bb = bb.at[0].set(0).at[NBLK].set(N_EDGES)
    lo = (bb[:NBLK] // 8) * 8
    hi = jnp.minimum(((bb[1:] + 7) // 8) * 8, N_EDGES)
    cnt_b = jnp.maximum(hi - lo, 0)

    b_idx = (jnp.arange(NT, dtype=jnp.int32)[:, None] * PH
             + jnp.arange(PH, dtype=jnp.int32)[None, :])
    meta = jnp.zeros((NT, 16), jnp.int32)
    meta = meta.at[:, 0:2 * PH:2].set(lo[b_idx])
    meta = meta.at[:, 1:2 * PH:2].set(cnt_b[b_idx])

    srcs_p = jnp.concatenate(
        [src_s, jnp.full((EB,), ZERO_ROW, jnp.int32)])
    dsts_p = jnp.concatenate([dst_s, jnp.zeros((EB,), jnp.int32)])

    g_ids = jnp.arange(N_GRAPHS, dtype=jnp.int32)
    offs = jnp.searchsorted(batch, g_ids, side="left").astype(jnp.int32)
    ends = jnp.searchsorted(batch, g_ids, side="right").astype(jnp.int32)
    cnts = ends - offs
    slot = jnp.arange(NSLOT, dtype=jnp.int32)
    sg = slot // MAX_NODES
    sp = slot % MAX_NODES
    pad_row = N_NODES + (slot % (NP - N_NODES))
    src_map = jnp.where(sp < cnts[sg], offs[sg] + sp, pad_row)

    xp = jnp.zeros((NP, DP), jnp.float32).at[:N_NODES, :D].set(x)
    W1p = jnp.zeros((L, DP, D2P), jnp.float32).at[:, :D, :2 * D].set(W1)
    b1p = jnp.zeros((L, 1, D2P), jnp.float32).at[:, 0, :2 * D].set(b1)
    W2p = jnp.zeros((L, D2P, DP), jnp.float32).at[:, :2 * D, :D].set(W2)
    b2p = jnp.zeros((L, 1, DP), jnp.float32).at[:, 0, :D].set(b2)
    fcWp = jnp.zeros((DP, H), jnp.float32).at[:D, :].set(fc_W)
    fcb2 = fc_b[None, :]
    zrows = jnp.zeros((ROWS_PH * DP,), jnp.float32)

    h = xp
    for i in range(L):
        agg = _agg(h, srcs_p, dsts_p, meta, zrows).reshape(NP, DP)
        h = _mlp(h, agg, W1p[i], b1p[i], W2p[i], b2p[i], relu_out=(i < L - 1))
    y = _fc(h, fcWp, fcb2)
    return _dense_out(src_map, y)


# speedup vs baseline: 1.6960x; 1.2824x over previous
import functools

import jax
import jax.numpy as jnp
from jax import lax
from jax.experimental import pallas as pl
from jax.experimental.pallas import tpu as pltpu
from jax.experimental.pallas import tpu_sc as plsc

N_NODES = 10000
N_EDGES = 160000
N_GRAPHS = 128
MAX_NODES = 256
D = 300
H = 768
L = 5

DP = 384
D2P = 608
NP = 10496
ZERO_ROW = 10200
EB = 64
NC = 2
NS = 16
NT = NC * NS
PH = 2
ROWS_PH = NP // (NT * PH)
NBLK = NT * PH
BLK = 328


def _agg_body(x_hbm, srcs_hbm, dsts_hbm, meta_hbm, zrows_hbm, agg_hbm,
              mvec, sidx, didx, gidxA, gidxB, ldixA, ldixB, stageA, stageB,
              acc, semA, semB):
    c = lax.axis_index("c")
    s = lax.axis_index("s")
    t = s * NC + c

    pltpu.sync_copy(meta_hbm.at[t], mvec)
    m = mvec[...]
    lane = lax.broadcasted_iota(jnp.int32, (16,), 0)
    zero16 = jnp.zeros((16,), jnp.int32)

    for p in range(PH):
        start = m[2 * p]
        cnt = m[2 * p + 1]
        qbase = (t * PH + p) * ROWS_PH

        pltpu.sync_copy(zrows_hbm, acc)

        nb = (cnt + EB - 1) // EB

        def stage_batch(b, gidx, ldix):
            off = pl.multiple_of(start + b * EB, 8)
            pltpu.sync_copy(srcs_hbm.at[pl.ds(off, EB)], sidx)
            pltpu.sync_copy(dsts_hbm.at[pl.ds(off, EB)], didx)
            for j in range(EB // 16):
                sv = sidx[pl.ds(j * 16, 16)]
                dv = didx[pl.ds(j * 16, 16)]
                eid = b * EB + j * 16 + lane
                ok = (eid < cnt) & (dv >= qbase) & (dv < qbase + ROWS_PH)
                gidx[pl.ds(j * 16, 16)] = jnp.where(ok, sv, ZERO_ROW + lane)
                ldix[pl.ds(j * 16, 16)] = jnp.where(ok, dv - qbase, 0)

        def gather_start(gidx, stage, sem):
            pltpu.make_async_copy(x_hbm.at[gidx], stage, sem).start()

        def gather_wait(gidx, stage, sem):
            pltpu.make_async_copy(x_hbm.at[gidx], stage, sem).wait()

        def accum(stage, ldix):
            def edge(e, carry2):
                lv = ldix[pl.ds(e, 16)]
                bcast0 = lax.gather(
                    lv, zero16[:, None],
                    lax.GatherDimensionNumbers(
                        offset_dims=(), collapsed_slice_dims=(0,),
                        start_index_map=(0,)),
                    slice_sizes=(1,),
                    mode=lax.GatherScatterMode.PROMISE_IN_BOUNDS)
                rbv = bcast0 * DP + lane
                for cc in range(DP // 16):
                    vals = stage[e, pl.ds(cc * 16, 16)]
                    plsc.addupdate_scatter(acc, [rbv + cc * 16], vals)
                return carry2

            lax.fori_loop(0, EB, edge, 0)

        @pl.when(nb > 0)
        def _():
            stage_batch(0, gidxA, ldixA)
            gather_start(gidxA, stageA, semA)

        def body(ii, carry):
            b0 = ii * 2

            @pl.when(b0 + 1 < nb)
            def _():
                stage_batch(b0 + 1, gidxB, ldixB)
                gather_start(gidxB, stageB, semB)

            gather_wait(gidxA, stageA, semA)
            accum(stageA, ldixA)

            @pl.when(b0 + 2 < nb)
            def _():
                stage_batch(b0 + 2, gidxA, ldixA)
                gather_start(gidxA, stageA, semA)

            @pl.when(b0 + 1 < nb)
            def _():
                gather_wait(gidxB, stageB, semB)
                accum(stageB, ldixB)

            return carry

        lax.fori_loop(0, (nb + 1) // 2, body, 0)
        pltpu.sync_copy(acc, agg_hbm.at[pl.ds(qbase * DP, ROWS_PH * DP)])


@functools.lru_cache(maxsize=1)
def _make_agg():
    return pl.kernel(
        _agg_body,
        out_type=jax.ShapeDtypeStruct((NP * DP,), jnp.float32),
        mesh=plsc.VectorSubcoreMesh(core_axis_name="c", subcore_axis_name="s"),
        compiler_params=pltpu.CompilerParams(needs_layout_passes=False),
        scratch_types=[
            pltpu.VMEM((16,), jnp.int32),
            pltpu.VMEM((EB,), jnp.int32),
            pltpu.VMEM((EB,), jnp.int32),
            pltpu.VMEM((EB,), jnp.int32),
            pltpu.VMEM((EB,), jnp.int32),
            pltpu.VMEM((EB + 16,), jnp.int32),
            pltpu.VMEM((EB + 16,), jnp.int32),
            pltpu.VMEM((EB, DP), jnp.float32),
            pltpu.VMEM((EB, DP), jnp.float32),
            pltpu.VMEM((ROWS_PH * DP,), jnp.float32),
            pltpu.SemaphoreType.DMA,
            pltpu.SemaphoreType.DMA,
        ],
    )


def _agg(h, srcs_p, dsts_p, meta, zrows):
    return _make_agg()(h, srcs_p, dsts_p, meta, zrows)


def _mlp_body(h_ref, a_ref, w1_ref, b1_ref, w2_ref, b2_ref, o_ref, *, relu_out):
    i = pl.program_id(0)
    z = h_ref[...] + a_ref[...]
    hid = jnp.maximum(
        jnp.dot(z, w1_ref[...], preferred_element_type=jnp.float32) + b1_ref[...],
        0.0)
    out = jnp.dot(hid, w2_ref[...], preferred_element_type=jnp.float32) + b2_ref[...]
    if relu_out:
        out = jnp.maximum(out, 0.0)
    rows = i * BLK + lax.broadcasted_iota(jnp.int32, (BLK, 1), 0)
    o_ref[...] = jnp.where(rows < N_NODES, out, 0.0)


def _mlp(h, agg, w1, b1, w2, b2, relu_out):
    return pl.pallas_call(
        functools.partial(_mlp_body, relu_out=relu_out),
        grid=(NP // BLK,),
        in_specs=[
            pl.BlockSpec((BLK, DP), lambda i: (i, 0)),
            pl.BlockSpec((BLK, DP), lambda i: (i, 0)),
            pl.BlockSpec((DP, D2P), lambda i: (0, 0)),
            pl.BlockSpec((1, D2P), lambda i: (0, 0)),
            pl.BlockSpec((D2P, DP), lambda i: (0, 0)),
            pl.BlockSpec((1, DP), lambda i: (0, 0)),
        ],
        out_specs=pl.BlockSpec((BLK, DP), lambda i: (i, 0)),
        out_shape=jax.ShapeDtypeStruct((NP, DP), jnp.float32),
    )(h, agg, w1, b1, w2, b2)


def _fc_body(h_ref, w_ref, b_ref, o_ref):
    o_ref[...] = (
        jnp.dot(h_ref[...], w_ref[...], preferred_element_type=jnp.float32)
        + b_ref[...])


def _fc(h, fc_w, fc_b2):
    return pl.pallas_call(
        _fc_body,
        grid=(NP // BLK,),
        in_specs=[
            pl.BlockSpec((BLK, DP), lambda i: (i, 0)),
            pl.BlockSpec((DP, H), lambda i: (0, 0)),
            pl.BlockSpec((1, H), lambda i: (0, 0)),
        ],
        out_specs=pl.BlockSpec((BLK, H), lambda i: (i, 0)),
        out_shape=jax.ShapeDtypeStruct((NP, H), jnp.float32),
    )(h, fc_w, fc_b2)


NSLOT = N_GRAPHS * MAX_NODES
SLOT_T = NSLOT // NT
OB = 64


def _batch_body(y_hbm, map_hbm, out_hbm, midx, stage, sem):
    c = lax.axis_index("c")
    s = lax.axis_index("s")
    t = s * NC + c

    def body(b, carry):
        base = pl.multiple_of(t * SLOT_T + b * OB, 8)
        pltpu.sync_copy(map_hbm.at[pl.ds(base, OB)], midx)
        pltpu.async_copy(y_hbm.at[midx], stage, sem).wait()
        pltpu.sync_copy(stage, out_hbm.at[pl.ds(base, OB)])
        return carry

    lax.fori_loop(0, SLOT_T // OB, body, 0)


@functools.lru_cache(maxsize=1)
def _make_batch():
    return pl.kernel(
        _batch_body,
        out_type=jax.ShapeDtypeStruct((NSLOT, H), jnp.float32),
        mesh=plsc.VectorSubcoreMesh(core_axis_name="c", subcore_axis_name="s"),
        compiler_params=pltpu.CompilerParams(needs_layout_passes=False),
        scratch_types=[
            pltpu.VMEM((OB,), jnp.int32),
            pltpu.VMEM((OB, H), jnp.float32),
            pltpu.SemaphoreType.DMA,
        ],
    )


def _dense_out(src_map, y):
    return _make_batch()(y, src_map).reshape(N_GRAPHS, MAX_NODES, H)


def kernel(x, edge_index, batch, W1, b1, W2, b2, fc_W, fc_b):
    src = edge_index[0].astype(jnp.int32)
    dst = edge_index[1].astype(jnp.int32)
    perm = jnp.argsort(dst)
    src_s = src[perm]
    dst_s = dst[perm]
    bb = jnp.searchsorted(
        dst_s, jnp.arange(NBLK + 1, dtype=jnp.int32) * ROWS_PH).astype(jnp.int32)
    bb = bb.at[0].set(0).at[NBLK].set(N_EDGES)
    lo = (bb[:NBLK] // 8) * 8
    hi = jnp.minimum(((bb[1:] + 7) // 8) * 8, N_EDGES)
    cnt_b = jnp.maximum(hi - lo, 0)

    b_idx = (jnp.arange(NT, dtype=jnp.int32)[:, None] * PH
             + jnp.arange(PH, dtype=jnp.int32)[None, :])
    meta = jnp.zeros((NT, 16), jnp.int32)
    meta = meta.at[:, 0:2 * PH:2].set(lo[b_idx])
    meta = meta.at[:, 1:2 * PH:2].set(cnt_b[b_idx])

    srcs_p = jnp.concatenate(
        [src_s, jnp.full((EB,), ZERO_ROW, jnp.int32)])
    dsts_p = jnp.concatenate([dst_s, jnp.zeros((EB,), jnp.int32)])

    g_ids = jnp.arange(N_GRAPHS, dtype=jnp.int32)
    offs = jnp.searchsorted(batch, g_ids, side="left").astype(jnp.int32)
    ends = jnp.searchsorted(batch, g_ids, side="right").astype(jnp.int32)
    cnts = ends - offs
    slot = jnp.arange(NSLOT, dtype=jnp.int32)
    sg = slot // MAX_NODES
    sp = slot % MAX_NODES
    pad_row = N_NODES + (slot % (NP - N_NODES))
    src_map = jnp.where(sp < cnts[sg], offs[sg] + sp, pad_row)

    xp = jnp.zeros((NP, DP), jnp.float32).at[:N_NODES, :D].set(x)
    W1p = jnp.zeros((L, DP, D2P), jnp.float32).at[:, :D, :2 * D].set(W1)
    b1p = jnp.zeros((L, 1, D2P), jnp.float32).at[:, 0, :2 * D].set(b1)
    W2p = jnp.zeros((L, D2P, DP), jnp.float32).at[:, :2 * D, :D].set(W2)
    b2p = jnp.zeros((L, 1, DP), jnp.float32).at[:, 0, :D].set(b2)
    fcWp = jnp.zeros((DP, H), jnp.float32).at[:D, :].set(fc_W)
    fcb2 = fc_b[None, :]
    zrows = jnp.zeros((ROWS_PH * DP,), jnp.float32)

    h = xp
    for i in range(L):
        agg = _agg(h, srcs_p, dsts_p, meta, zrows).reshape(NP, DP)
        h = _mlp(h, agg, W1p[i], b1p[i], W2p[i], b2p[i], relu_out=(i < L - 1))
    y = _fc(h, fcWp, fcb2)
    return _dense_out(src_map, y)


# speedup vs baseline: 1.7161x; 1.0119x over previous
import functools

import jax
import jax.numpy as jnp
from jax import lax
from jax.experimental import pallas as pl
from jax.experimental.pallas import tpu as pltpu
from jax.experimental.pallas import tpu_sc as plsc

N_NODES = 10000
N_EDGES = 160000
N_GRAPHS = 128
MAX_NODES = 256
D = 300
H = 768
L = 5

DP = 384
D2P = 608
NP = 10496
ZERO_ROW = 10200
EB = 64
NC = 2
NS = 16
NT = NC * NS
PH = 2
ROWS_PH = NP // (NT * PH)
NBLK = NT * PH
BLK = 328


def _agg_body(x_hbm, srcs_hbm, dsts_hbm, meta_hbm, zrows_hbm, agg_hbm,
              mvec, sidx, didx, gidxA, gidxB, ldixA, ldixB, stageA, stageB,
              acc, semA, semB):
    c = lax.axis_index("c")
    s = lax.axis_index("s")
    t = s * NC + c

    pltpu.sync_copy(meta_hbm.at[t], mvec)
    m = mvec[...]
    lane = lax.broadcasted_iota(jnp.int32, (16,), 0)
    zero16 = jnp.zeros((16,), jnp.int32)

    for p in range(PH):
        start = m[2 * p]
        cnt = m[2 * p + 1]
        qbase = (t * PH + p) * ROWS_PH

        pltpu.sync_copy(zrows_hbm, acc)

        nb = (cnt + EB - 1) // EB

        def stage_batch(b, gidx, ldix):
            off = pl.multiple_of(start + b * EB, 8)
            pltpu.sync_copy(srcs_hbm.at[pl.ds(off, EB)], sidx)
            pltpu.sync_copy(dsts_hbm.at[pl.ds(off, EB)], didx)
            for j in range(EB // 16):
                sv = sidx[pl.ds(j * 16, 16)]
                dv = didx[pl.ds(j * 16, 16)]
                eid = b * EB + j * 16 + lane
                ok = (eid < cnt) & (dv >= qbase) & (dv < qbase + ROWS_PH)
                gidx[pl.ds(j * 16, 16)] = jnp.where(ok, sv, ZERO_ROW + lane)
                ldix[pl.ds(j * 16, 16)] = jnp.where(ok, dv - qbase, 0)

        def gather_start(gidx, stage, sem):
            pltpu.make_async_copy(x_hbm.at[gidx], stage, sem).start()

        def gather_wait(gidx, stage, sem):
            pltpu.make_async_copy(x_hbm.at[gidx], stage, sem).wait()

        def accum(stage, ldix):
            def grp16(k, carry2):
                lv = ldix[pl.ds(k * 16, 16)]
                for jj in range(16):
                    bcast = lax.gather(
                        lv, jnp.full((16, 1), jj, jnp.int32),
                        lax.GatherDimensionNumbers(
                            offset_dims=(), collapsed_slice_dims=(0,),
                            start_index_map=(0,)),
                        slice_sizes=(1,),
                        mode=lax.GatherScatterMode.PROMISE_IN_BOUNDS)
                    rbv = bcast * DP + lane
                    row = k * 16 + jj
                    for cc in range(DP // 16):
                        vals = stage[row, pl.ds(cc * 16, 16)]
                        plsc.addupdate_scatter(acc, [rbv + cc * 16], vals)
                return carry2

            lax.fori_loop(0, EB // 16, grp16, 0)

        @pl.when(nb > 0)
        def _():
            stage_batch(0, gidxA, ldixA)
            gather_start(gidxA, stageA, semA)

        def body(ii, carry):
            b0 = ii * 2

            @pl.when(b0 + 1 < nb)
            def _():
                stage_batch(b0 + 1, gidxB, ldixB)
                gather_start(gidxB, stageB, semB)

            gather_wait(gidxA, stageA, semA)
            accum(stageA, ldixA)

            @pl.when(b0 + 2 < nb)
            def _():
                stage_batch(b0 + 2, gidxA, ldixA)
                gather_start(gidxA, stageA, semA)

            @pl.when(b0 + 1 < nb)
            def _():
                gather_wait(gidxB, stageB, semB)
                accum(stageB, ldixB)

            return carry

        lax.fori_loop(0, (nb + 1) // 2, body, 0)
        pltpu.sync_copy(acc, agg_hbm.at[pl.ds(qbase * DP, ROWS_PH * DP)])


@functools.lru_cache(maxsize=1)
def _make_agg():
    return pl.kernel(
        _agg_body,
        out_type=jax.ShapeDtypeStruct((NP * DP,), jnp.float32),
        mesh=plsc.VectorSubcoreMesh(core_axis_name="c", subcore_axis_name="s"),
        compiler_params=pltpu.CompilerParams(needs_layout_passes=False),
        scratch_types=[
            pltpu.VMEM((16,), jnp.int32),
            pltpu.VMEM((EB,), jnp.int32),
            pltpu.VMEM((EB,), jnp.int32),
            pltpu.VMEM((EB,), jnp.int32),
            pltpu.VMEM((EB,), jnp.int32),
            pltpu.VMEM((EB + 16,), jnp.int32),
            pltpu.VMEM((EB + 16,), jnp.int32),
            pltpu.VMEM((EB, DP), jnp.float32),
            pltpu.VMEM((EB, DP), jnp.float32),
            pltpu.VMEM((ROWS_PH * DP,), jnp.float32),
            pltpu.SemaphoreType.DMA,
            pltpu.SemaphoreType.DMA,
        ],
    )


def _agg(h, srcs_p, dsts_p, meta, zrows):
    return _make_agg()(h, srcs_p, dsts_p, meta, zrows)


def _mlp_body(h_ref, a_ref, w1_ref, b1_ref, w2_ref, b2_ref, o_ref, *, relu_out):
    i = pl.program_id(0)
    z = h_ref[...] + a_ref[...]
    hid = jnp.maximum(
        jnp.dot(z, w1_ref[...], preferred_element_type=jnp.float32) + b1_ref[...],
        0.0)
    out = jnp.dot(hid, w2_ref[...], preferred_element_type=jnp.float32) + b2_ref[...]
    if relu_out:
        out = jnp.maximum(out, 0.0)
    rows = i * BLK + lax.broadcasted_iota(jnp.int32, (BLK, 1), 0)
    o_ref[...] = jnp.where(rows < N_NODES, out, 0.0)


def _mlp(h, agg, w1, b1, w2, b2, relu_out):
    return pl.pallas_call(
        functools.partial(_mlp_body, relu_out=relu_out),
        grid=(NP // BLK,),
        in_specs=[
            pl.BlockSpec((BLK, DP), lambda i: (i, 0)),
            pl.BlockSpec((BLK, DP), lambda i: (i, 0)),
            pl.BlockSpec((DP, D2P), lambda i: (0, 0)),
            pl.BlockSpec((1, D2P), lambda i: (0, 0)),
            pl.BlockSpec((D2P, DP), lambda i: (0, 0)),
            pl.BlockSpec((1, DP), lambda i: (0, 0)),
        ],
        out_specs=pl.BlockSpec((BLK, DP), lambda i: (i, 0)),
        out_shape=jax.ShapeDtypeStruct((NP, DP), jnp.float32),
    )(h, agg, w1, b1, w2, b2)


def _fc_body(h_ref, w_ref, b_ref, o_ref):
    o_ref[...] = (
        jnp.dot(h_ref[...], w_ref[...], preferred_element_type=jnp.float32)
        + b_ref[...])


def _fc(h, fc_w, fc_b2):
    return pl.pallas_call(
        _fc_body,
        grid=(NP // BLK,),
        in_specs=[
            pl.BlockSpec((BLK, DP), lambda i: (i, 0)),
            pl.BlockSpec((DP, H), lambda i: (0, 0)),
            pl.BlockSpec((1, H), lambda i: (0, 0)),
        ],
        out_specs=pl.BlockSpec((BLK, H), lambda i: (i, 0)),
        out_shape=jax.ShapeDtypeStruct((NP, H), jnp.float32),
    )(h, fc_w, fc_b2)


NSLOT = N_GRAPHS * MAX_NODES
SLOT_T = NSLOT // NT
OB = 64


def _batch_body(y_hbm, map_hbm, out_hbm, midx, stage, sem):
    c = lax.axis_index("c")
    s = lax.axis_index("s")
    t = s * NC + c

    def body(b, carry):
        base = pl.multiple_of(t * SLOT_T + b * OB, 8)
        pltpu.sync_copy(map_hbm.at[pl.ds(base, OB)], midx)
        pltpu.async_copy(y_hbm.at[midx], stage, sem).wait()
        pltpu.sync_copy(stage, out_hbm.at[pl.ds(base, OB)])
        return carry

    lax.fori_loop(0, SLOT_T // OB, body, 0)


@functools.lru_cache(maxsize=1)
def _make_batch():
    return pl.kernel(
        _batch_body,
        out_type=jax.ShapeDtypeStruct((NSLOT, H), jnp.float32),
        mesh=plsc.VectorSubcoreMesh(core_axis_name="c", subcore_axis_name="s"),
        compiler_params=pltpu.CompilerParams(needs_layout_passes=False),
        scratch_types=[
            pltpu.VMEM((OB,), jnp.int32),
            pltpu.VMEM((OB, H), jnp.float32),
            pltpu.SemaphoreType.DMA,
        ],
    )


def _dense_out(src_map, y):
    return _make_batch()(y, src_map).reshape(N_GRAPHS, MAX_NODES, H)


def kernel(x, edge_index, batch, W1, b1, W2, b2, fc_W, fc_b):
    src = edge_index[0].astype(jnp.int32)
    dst = edge_index[1].astype(jnp.int32)
    perm = jnp.argsort(dst)
    src_s = src[perm]
    dst_s = dst[perm]
    bb = jnp.searchsorted(
        dst_s, jnp.arange(NBLK + 1, dtype=jnp.int32) * ROWS_PH).astype(jnp.int32)
    bb = bb.at[0].set(0).at[NBLK].set(N_EDGES)
    lo = (bb[:NBLK] // 8) * 8
    hi = jnp.minimum(((bb[1:] + 7) // 8) * 8, N_EDGES)
    cnt_b = jnp.maximum(hi - lo, 0)

    b_idx = (jnp.arange(NT, dtype=jnp.int32)[:, None] * PH
             + jnp.arange(PH, dtype=jnp.int32)[None, :])
    meta = jnp.zeros((NT, 16), jnp.int32)
    meta = meta.at[:, 0:2 * PH:2].set(lo[b_idx])
    meta = meta.at[:, 1:2 * PH:2].set(cnt_b[b_idx])

    srcs_p = jnp.concatenate(
        [src_s, jnp.full((EB,), ZERO_ROW, jnp.int32)])
    dsts_p = jnp.concatenate([dst_s, jnp.zeros((EB,), jnp.int32)])

    g_ids = jnp.arange(N_GRAPHS, dtype=jnp.int32)
    offs = jnp.searchsorted(batch, g_ids, side="left").astype(jnp.int32)
    ends = jnp.searchsorted(batch, g_ids, side="right").astype(jnp.int32)
    cnts = ends - offs
    slot = jnp.arange(NSLOT, dtype=jnp.int32)
    sg = slot // MAX_NODES
    sp = slot % MAX_NODES
    pad_row = N_NODES + (slot % (NP - N_NODES))
    src_map = jnp.where(sp < cnts[sg], offs[sg] + sp, pad_row)

    xp = jnp.zeros((NP, DP), jnp.float32).at[:N_NODES, :D].set(x)
    W1p = jnp.zeros((L, DP, D2P), jnp.float32).at[:, :D, :2 * D].set(W1)
    b1p = jnp.zeros((L, 1, D2P), jnp.float32).at[:, 0, :2 * D].set(b1)
    W2p = jnp.zeros((L, D2P, DP), jnp.float32).at[:, :2 * D, :D].set(W2)
    b2p = jnp.zeros((L, 1, DP), jnp.float32).at[:, 0, :D].set(b2)
    fcWp = jnp.zeros((DP, H), jnp.float32).at[:D, :].set(fc_W)
    fcb2 = fc_b[None, :]
    zrows = jnp.zeros((ROWS_PH * DP,), jnp.float32)

    h = xp
    for i in range(L):
        agg = _agg(h, srcs_p, dsts_p, meta, zrows).reshape(NP, DP)
        h = _mlp(h, agg, W1p[i], b1p[i], W2p[i], b2p[i], relu_out=(i < L - 1))
    y = _fc(h, fcWp, fcb2)
    return _dense_out(src_map, y)
